# baseline (speedup 1.0000x reference)
import jax
import jax.numpy as jnp
import numpy as np
from jax import lax
from jax.experimental import pallas as pl
from jax.experimental.pallas import tpu as pltpu
from jax.experimental.pallas import tpu_sc as plsc

_B = 64
_EMBED = 768
_NVIS = 256
_NMASK = 768
_NROWS = 1025
_NW = 32
_RB = 32
_NBLK = (_B * _NROWS) // _RB
_KMAX = (_NBLK + _NW - 1) // _NW
_NPAD = 1040


def _build_maps(sidx_ref, midx_ref, src_ref):
    def init(i, _):
        src_ref[i] = -1
        return 0

    lax.fori_loop(0, _NPAD, init, 0)
    src_ref[0] = 0

    def samp(j, _):
        src_ref[sidx_ref[j]] = j + 1
        return 0

    lax.fori_loop(0, _NVIS, samp, 0)

    def msk(j, _):
        src_ref[midx_ref[j]] = -2
        return 0

    lax.fori_loop(0, _NMASK, msk, 0)


def _sc_body(x_hbm, src_hbm, w0_hbm, b_hbm, out_hbm,
             srcb, gxb, slotb, xstage, stg0, stg1, wbuf, bbuf, semg, semo):
    cid = lax.axis_index("c")
    sid = lax.axis_index("s")
    wid = sid * 2 + cid
    pltpu.sync_copy(src_hbm, srcb)
    pltpu.sync_copy(w0_hbm, wbuf)
    pltpu.sync_copy(b_hbm.at[pl.ds(0, 16)], bbuf)
    acc = jnp.zeros((16,), jnp.float32)
    for k in range(_EMBED // 16):
        acc = acc + wbuf[pl.ds(k * 16, 16)]
    wsum = acc[0]
    for k in range(1, 16):
        wsum = wsum + acc[k]
    b0 = bbuf[...][0]
    s_val = wsum * np.float32(127.0 / 255.0) + b0
    sv = jnp.full((16,), s_val, jnp.float32)
    zv = jnp.zeros((16,), jnp.float32)

    def block(blk, stg, oc):
        q0 = blk * _RB

        @pl.when(oc >= 2)
        def _():
            pltpu.make_async_copy(stg, out_hbm.at[pl.ds(0, _RB)], semo).wait()

        def row(i, cur):
            q = q0 + i
            r = lax.rem(q, _NROWS)
            bb = q // _NROWS
            src_i = srcb[pl.ds(r, 16)][0]

            @pl.when(src_i >= 0)
            def _():
                gxb[pl.ds(cur, 16)] = jnp.full(
                    (16,), bb * (1 + _NVIS), jnp.int32
                ) + src_i
                slotb[pl.ds(cur, 16)] = jnp.full((16,), i, jnp.int32)

            @pl.when(src_i == -1)
            def _():
                for c in range(_EMBED // 16):
                    stg[i, pl.ds(c * 16, 16)] = zv

            @pl.when(src_i == -2)
            def _():
                for c in range(_EMBED // 16):
                    stg[i, pl.ds(c * 16, 16)] = sv

            return cur + (src_i >= 0).astype(jnp.int32)

        cur = lax.fori_loop(0, _RB, row, 0)

        nch = (cur + 7) // 8

        def fire(c, _):
            pltpu.async_copy(
                x_hbm.at[gxb.at[pl.ds(c * 8, 8)]],
                xstage.at[pl.ds(c * 8, 8)],
                semg,
            )
            return 0

        lax.fori_loop(0, nch, fire, 0)

        def draing(c, _):
            pltpu.make_async_copy(
                x_hbm.at[gxb.at[pl.ds(0, 8)]], xstage.at[pl.ds(0, 8)], semg
            ).wait()
            return 0

        lax.fori_loop(0, nch, draing, 0)

        def redis(m, _):
            slot = slotb[pl.ds(m, 16)][0]
            for c in range(_EMBED // 16):
                stg[slot, pl.ds(c * 16, 16)] = xstage[m, pl.ds(c * 16, 16)]
            return 0

        lax.fori_loop(0, cur, redis, 0)

        pltpu.async_copy(stg, out_hbm.at[pl.ds(q0, _RB)], semo)

    def blk_body(k, oc):
        blk = wid + _NW * k

        @pl.when(jnp.logical_and(blk < _NBLK, lax.rem(k, 2) == 0))
        def _():
            block(blk, stg0, oc)

        @pl.when(jnp.logical_and(blk < _NBLK, lax.rem(k, 2) == 1))
        def _():
            block(blk, stg1, oc)

        return jnp.where(blk < _NBLK, jnp.minimum(oc + 1, 2), oc)

    oc = lax.fori_loop(0, _KMAX, blk_body, 0)

    def draino(j, c):
        pltpu.make_async_copy(stg0, out_hbm.at[pl.ds(0, _RB)], semo).wait()
        return c

    lax.fori_loop(0, oc, draino, 0)


def kernel(x, sample_index, mask_index, W, b):
    src = pl.pallas_call(
        _build_maps,
        in_specs=[
            pl.BlockSpec(memory_space=pltpu.SMEM),
            pl.BlockSpec(memory_space=pltpu.SMEM),
        ],
        out_specs=pl.BlockSpec(memory_space=pltpu.SMEM),
        out_shape=jax.ShapeDtypeStruct((_NPAD,), jnp.int32),
    )(sample_index, mask_index)

    x2d = jnp.reshape(x, (_B * (1 + _NVIS), _EMBED))
    w0 = jnp.reshape(W[0], (_EMBED,))

    mesh = plsc.VectorSubcoreMesh(core_axis_name="c", subcore_axis_name="s")
    out2 = pl.kernel(
        _sc_body,
        out_type=jax.ShapeDtypeStruct((_B * _NROWS, _EMBED), jnp.float32),
        mesh=mesh,
        scratch_types=[
            pltpu.VMEM((_NPAD,), jnp.int32),
            pltpu.VMEM((64,), jnp.int32),
            pltpu.VMEM((64,), jnp.int32),
            pltpu.VMEM((_RB, _EMBED), jnp.float32),
            pltpu.VMEM((_RB, _EMBED), jnp.float32),
            pltpu.VMEM((_RB, _EMBED), jnp.float32),
            pltpu.VMEM((_EMBED,), jnp.float32),
            pltpu.VMEM((16,), jnp.float32),
            pltpu.SemaphoreType.DMA,
            pltpu.SemaphoreType.DMA,
        ],
    )(x2d, src, w0, b)

    return jnp.reshape(out2, (_B, _NROWS, _EMBED))


# speedup vs baseline: 1.3274x; 1.3274x over previous
import jax
import jax.numpy as jnp
import numpy as np
from jax import lax
from jax.experimental import pallas as pl
from jax.experimental.pallas import tpu as pltpu
from jax.experimental.pallas import tpu_sc as plsc

_B = 64
_EMBED = 768
_NVIS = 256
_NMASK = 768
_NROWS = 1025
_NW = 32
_RB = 32
_TPB = 32
_KMAX = (_B * _TPB) // _NW
_NPAD = 1040


def _build_maps(sidx_ref, midx_ref, src_ref):
    def init(i, _):
        src_ref[i] = -1
        return 0

    lax.fori_loop(0, _NPAD, init, 0)
    src_ref[0] = 0

    def samp(j, _):
        src_ref[sidx_ref[j]] = j + 1
        return 0

    lax.fori_loop(0, _NVIS, samp, 0)

    def msk(j, _):
        src_ref[midx_ref[j]] = -2
        return 0

    lax.fori_loop(0, _NMASK, msk, 0)


def _sc_body(x_hbm, src_hbm, w0_hbm, b_hbm, out_hbm,
             srcb, gxb, slotb, xstage, stg0, stg1, wbuf, bbuf, semg, semo):
    cid = lax.axis_index("c")
    sid = lax.axis_index("s")
    wid = sid * 2 + cid
    pltpu.sync_copy(src_hbm, srcb)
    pltpu.sync_copy(w0_hbm, wbuf)
    pltpu.sync_copy(b_hbm.at[pl.ds(0, 16)], bbuf)
    acc = jnp.zeros((16,), jnp.float32)
    for k in range(_EMBED // 16):
        acc = acc + wbuf[pl.ds(k * 16, 16)]
    wsum = acc[0]
    for k in range(1, 16):
        wsum = wsum + acc[k]
    b0 = bbuf[...][0]
    s_val = wsum * np.float32(127.0 / 255.0) + b0
    sv = jnp.full((16,), s_val, jnp.float32)
    zv = jnp.zeros((16,), jnp.float32)

    def block(blkid, stg, oc):
        bb = blkid // _TPB
        t = lax.rem(blkid, _TPB)
        r0 = t * _RB

        @pl.when(oc >= 2)
        def _():
            pltpu.make_async_copy(
                stg, out_hbm.at[0, pl.ds(0, _RB)], semo
            ).wait()

        def row(i, cur):
            src_i = srcb[pl.ds(r0 + i, 16)][0]

            @pl.when(src_i >= 0)
            def _():
                gxb[pl.ds(cur, 16)] = jnp.full(
                    (16,), bb * (1 + _NVIS), jnp.int32
                ) + src_i
                slotb[pl.ds(cur, 16)] = jnp.full((16,), i, jnp.int32)

            @pl.when(src_i == -1)
            def _():
                for c in range(_EMBED // 16):
                    stg[i, pl.ds(c * 16, 16)] = zv

            @pl.when(src_i == -2)
            def _():
                for c in range(_EMBED // 16):
                    stg[i, pl.ds(c * 16, 16)] = sv

            return cur + (src_i >= 0).astype(jnp.int32)

        cur = lax.fori_loop(0, _RB, row, 0)

        nch = (cur + 7) // 8

        def fire(c, _):
            pltpu.async_copy(
                x_hbm.at[gxb.at[pl.ds(c * 8, 8)]],
                xstage.at[pl.ds(c * 8, 8)],
                semg,
            )
            return 0

        lax.fori_loop(0, nch, fire, 0)

        def draing(c, _):
            pltpu.make_async_copy(
                x_hbm.at[gxb.at[pl.ds(0, 8)]], xstage.at[pl.ds(0, 8)], semg
            ).wait()
            return 0

        lax.fori_loop(0, nch, draing, 0)

        def redis(m, _):
            slot = slotb[pl.ds(m, 16)][0]
            for c in range(_EMBED // 16):
                stg[slot, pl.ds(c * 16, 16)] = xstage[m, pl.ds(c * 16, 16)]
            return 0

        lax.fori_loop(0, cur, redis, 0)

        pltpu.async_copy(stg, out_hbm.at[bb, pl.ds(r0, _RB)], semo)

    def blk_body(k, oc):
        blkid = wid + _NW * k

        @pl.when(lax.rem(k, 2) == 0)
        def _():
            block(blkid, stg0, oc)

        @pl.when(lax.rem(k, 2) == 1)
        def _():
            block(blkid, stg1, oc)

        return jnp.minimum(oc + 1, 2)

    oc = lax.fori_loop(0, _KMAX, blk_body, 0)

    def draino(j, c):
        pltpu.make_async_copy(stg0, out_hbm.at[0, pl.ds(0, _RB)], semo).wait()
        return c

    lax.fori_loop(0, oc, draino, 0)


def _tail(src_ref, dec_hbm, x_hbm, w0_ref, b_ref, o_hbm, stg, sem):
    src_t = src_ref[_NROWS - 1]
    s_val = jnp.sum(w0_ref[...]) * np.float32(127.0 / 255.0) + b_ref[0, 0]

    @pl.when(src_t >= 0)
    def _():
        cp = pltpu.make_async_copy(
            x_hbm.at[:, pl.ds(jnp.maximum(src_t, 0), 1)], stg, sem
        )
        cp.start()
        cp.wait()

    @pl.when(src_t == -1)
    def _():
        stg[...] = jnp.zeros((_B, 1, _EMBED), jnp.float32)

    @pl.when(src_t == -2)
    def _():
        stg[...] = jnp.full((_B, 1, _EMBED), s_val)

    cp = pltpu.make_async_copy(
        stg, o_hbm.at[:, pl.ds(_NROWS - 1, 1)], sem
    )
    cp.start()
    cp.wait()


def kernel(x, sample_index, mask_index, W, b):
    src = pl.pallas_call(
        _build_maps,
        in_specs=[
            pl.BlockSpec(memory_space=pltpu.SMEM),
            pl.BlockSpec(memory_space=pltpu.SMEM),
        ],
        out_specs=pl.BlockSpec(memory_space=pltpu.SMEM),
        out_shape=jax.ShapeDtypeStruct((_NPAD,), jnp.int32),
    )(sample_index, mask_index)

    x2d = jnp.reshape(x, (_B * (1 + _NVIS), _EMBED))
    w0 = jnp.reshape(W[0], (_EMBED,))

    mesh = plsc.VectorSubcoreMesh(core_axis_name="c", subcore_axis_name="s")
    dec = pl.kernel(
        _sc_body,
        out_type=jax.ShapeDtypeStruct((_B, _NROWS, _EMBED), jnp.float32),
        mesh=mesh,
        scratch_types=[
            pltpu.VMEM((_NPAD,), jnp.int32),
            pltpu.VMEM((64,), jnp.int32),
            pltpu.VMEM((64,), jnp.int32),
            pltpu.VMEM((_RB, _EMBED), jnp.float32),
            pltpu.VMEM((_RB, _EMBED), jnp.float32),
            pltpu.VMEM((_RB, _EMBED), jnp.float32),
            pltpu.VMEM((_EMBED,), jnp.float32),
            pltpu.VMEM((16,), jnp.float32),
            pltpu.SemaphoreType.DMA,
            pltpu.SemaphoreType.DMA,
        ],
    )(x2d, src, w0, b)

    w0r = jnp.reshape(W[0], (1, _EMBED))
    b2 = jnp.reshape(b, (1, _EMBED))
    out = pl.pallas_call(
        _tail,
        in_specs=[
            pl.BlockSpec(memory_space=pltpu.SMEM),
            pl.BlockSpec(memory_space=pl.ANY),
            pl.BlockSpec(memory_space=pl.ANY),
            pl.BlockSpec((1, _EMBED), lambda: (0, 0)),
            pl.BlockSpec((1, _EMBED), lambda: (0, 0)),
        ],
        out_specs=pl.BlockSpec(memory_space=pl.ANY),
        out_shape=jax.ShapeDtypeStruct((_B, _NROWS, _EMBED), jnp.float32),
        input_output_aliases={1: 0},
        scratch_shapes=[
            pltpu.VMEM((_B, 1, _EMBED), jnp.float32),
            pltpu.SemaphoreType.DMA,
        ],
    )(src, dec, x, w0r, b2)

    return out


# speedup vs baseline: 1.3328x; 1.0041x over previous
import jax
import jax.numpy as jnp
import numpy as np
from jax import lax
from jax.experimental import pallas as pl
from jax.experimental.pallas import tpu as pltpu
from jax.experimental.pallas import tpu_sc as plsc

_B = 64
_EMBED = 768
_NVIS = 256
_NMASK = 768
_NROWS = 1025
_NW = 32
_RB = 32
_TPB = 32
_KMAX = (_B * _TPB) // _NW
_NPAD = 1040


def _build_maps(sidx_ref, midx_ref, src_ref):
    def init(i, _):
        src_ref[i] = -1
        return 0

    lax.fori_loop(0, _NPAD, init, 0)
    src_ref[0] = 0

    def samp(j, _):
        src_ref[sidx_ref[j]] = j + 1
        return 0

    lax.fori_loop(0, _NVIS, samp, 0)

    def msk(j, _):
        src_ref[midx_ref[j]] = -2
        return 0

    lax.fori_loop(0, _NMASK, msk, 0)


def _sc_body(x_hbm, src_hbm, w0_hbm, b_hbm, out_hbm,
             srcb, gxb, slotb, xstage, gxb2, slotb2, xstage2,
             stg0, stg1, wbuf, bbuf, semg, semg2, semo):
    cid = lax.axis_index("c")
    sid = lax.axis_index("s")
    wid = sid * 2 + cid
    pltpu.sync_copy(src_hbm, srcb)
    pltpu.sync_copy(w0_hbm, wbuf)
    pltpu.sync_copy(b_hbm.at[pl.ds(0, 16)], bbuf)
    acc = jnp.zeros((16,), jnp.float32)
    for k in range(_EMBED // 16):
        acc = acc + wbuf[pl.ds(k * 16, 16)]
    wsum = acc[0]
    for k in range(1, 16):
        wsum = wsum + acc[k]
    b0 = bbuf[...][0]
    s_val = wsum * np.float32(127.0 / 255.0) + b0
    sv = jnp.full((16,), s_val, jnp.float32)
    zv = jnp.zeros((16,), jnp.float32)

    def start_block(k, stg, gxbk, slotbk, xstagek, sem):
        blkid = wid + _NW * k
        bb = blkid // _TPB
        r0 = lax.rem(blkid, _TPB) * _RB

        def row(i, cur):
            src_i = srcb[pl.ds(r0 + i, 16)][0]

            @pl.when(src_i >= 0)
            def _():
                gxbk[pl.ds(cur, 16)] = jnp.full(
                    (16,), bb * (1 + _NVIS), jnp.int32
                ) + src_i
                slotbk[pl.ds(cur, 16)] = jnp.full((16,), i, jnp.int32)

            @pl.when(src_i == -1)
            def _():
                for c in range(_EMBED // 16):
                    stg[i, pl.ds(c * 16, 16)] = zv

            @pl.when(src_i == -2)
            def _():
                for c in range(_EMBED // 16):
                    stg[i, pl.ds(c * 16, 16)] = sv

            return cur + (src_i >= 0).astype(jnp.int32)

        cur = lax.fori_loop(0, _RB, row, 0)
        nch = (cur + 7) // 8

        def fire(c, _):
            pltpu.async_copy(
                x_hbm.at[gxbk.at[pl.ds(c * 8, 8)]],
                xstagek.at[pl.ds(c * 8, 8)],
                sem,
            )
            return 0

        lax.fori_loop(0, nch, fire, 0)
        return cur

    def finish_block(k, stg, gxbk, slotbk, xstagek, cur, sem):
        blkid = wid + _NW * k
        bb = blkid // _TPB
        r0 = lax.rem(blkid, _TPB) * _RB
        nch = (cur + 7) // 8

        def draing(c, _):
            pltpu.make_async_copy(
                x_hbm.at[gxbk.at[pl.ds(0, 8)]], xstagek.at[pl.ds(0, 8)], sem
            ).wait()
            return 0

        lax.fori_loop(0, nch, draing, 0)

        def redis(m, _):
            slot = slotbk[pl.ds(m, 16)][0]
            for c in range(_EMBED // 16):
                stg[slot, pl.ds(c * 16, 16)] = xstagek[m, pl.ds(c * 16, 16)]
            return 0

        lax.fori_loop(0, cur, redis, 0)

        pltpu.async_copy(stg, out_hbm.at[bb, pl.ds(r0, _RB)], semo)

    def outwait():
        pltpu.make_async_copy(stg0, out_hbm.at[0, pl.ds(0, _RB)], semo).wait()

    def pair_body(kk, cur_prev):
        k0 = 2 * kk
        k1 = k0 + 1

        @pl.when(kk >= 1)
        def _():
            outwait()

        cur0 = start_block(k0, stg0, gxb, slotb, xstage, semg)

        @pl.when(kk >= 1)
        def _():
            finish_block(k0 - 1, stg1, gxb2, slotb2, xstage2, cur_prev, semg2)
            outwait()

        cur1 = start_block(k1, stg1, gxb2, slotb2, xstage2, semg2)
        finish_block(k0, stg0, gxb, slotb, xstage, cur0, semg)
        return cur1

    cur_last = lax.fori_loop(0, _KMAX // 2, pair_body, 0)
    finish_block(_KMAX - 1, stg1, gxb2, slotb2, xstage2, cur_last, semg2)
    outwait()
    outwait()


def _tail(src_ref, dec_hbm, x_hbm, w0_ref, b_ref, o_hbm, stg, sem):
    src_t = src_ref[_NROWS - 1]
    s_val = jnp.sum(w0_ref[...]) * np.float32(127.0 / 255.0) + b_ref[0, 0]

    @pl.when(src_t >= 0)
    def _():
        cp = pltpu.make_async_copy(
            x_hbm.at[:, pl.ds(jnp.maximum(src_t, 0), 1)], stg, sem
        )
        cp.start()
        cp.wait()

    @pl.when(src_t == -1)
    def _():
        stg[...] = jnp.zeros((_B, 1, _EMBED), jnp.float32)

    @pl.when(src_t == -2)
    def _():
        stg[...] = jnp.full((_B, 1, _EMBED), s_val)

    cp = pltpu.make_async_copy(
        stg, o_hbm.at[:, pl.ds(_NROWS - 1, 1)], sem
    )
    cp.start()
    cp.wait()


def kernel(x, sample_index, mask_index, W, b):
    src = pl.pallas_call(
        _build_maps,
        in_specs=[
            pl.BlockSpec(memory_space=pltpu.SMEM),
            pl.BlockSpec(memory_space=pltpu.SMEM),
        ],
        out_specs=pl.BlockSpec(memory_space=pltpu.SMEM),
        out_shape=jax.ShapeDtypeStruct((_NPAD,), jnp.int32),
    )(sample_index, mask_index)

    x2d = jnp.reshape(x, (_B * (1 + _NVIS), _EMBED))
    w0 = jnp.reshape(W[0], (_EMBED,))

    mesh = plsc.VectorSubcoreMesh(core_axis_name="c", subcore_axis_name="s")
    dec = pl.kernel(
        _sc_body,
        out_type=jax.ShapeDtypeStruct((_B, _NROWS, _EMBED), jnp.float32),
        mesh=mesh,
        scratch_types=[
            pltpu.VMEM((_NPAD,), jnp.int32),
            pltpu.VMEM((64,), jnp.int32),
            pltpu.VMEM((64,), jnp.int32),
            pltpu.VMEM((_RB, _EMBED), jnp.float32),
            pltpu.VMEM((64,), jnp.int32),
            pltpu.VMEM((64,), jnp.int32),
            pltpu.VMEM((_RB, _EMBED), jnp.float32),
            pltpu.VMEM((_RB, _EMBED), jnp.float32),
            pltpu.VMEM((_RB, _EMBED), jnp.float32),
            pltpu.VMEM((_EMBED,), jnp.float32),
            pltpu.VMEM((16,), jnp.float32),
            pltpu.SemaphoreType.DMA,
            pltpu.SemaphoreType.DMA,
            pltpu.SemaphoreType.DMA,
        ],
    )(x2d, src, w0, b)

    w0r = jnp.reshape(W[0], (1, _EMBED))
    b2 = jnp.reshape(b, (1, _EMBED))
    out = pl.pallas_call(
        _tail,
        in_specs=[
            pl.BlockSpec(memory_space=pltpu.SMEM),
            pl.BlockSpec(memory_space=pl.ANY),
            pl.BlockSpec(memory_space=pl.ANY),
            pl.BlockSpec((1, _EMBED), lambda: (0, 0)),
            pl.BlockSpec((1, _EMBED), lambda: (0, 0)),
        ],
        out_specs=pl.BlockSpec(memory_space=pl.ANY),
        out_shape=jax.ShapeDtypeStruct((_B, _NROWS, _EMBED), jnp.float32),
        input_output_aliases={1: 0},
        scratch_shapes=[
            pltpu.VMEM((_B, 1, _EMBED), jnp.float32),
            pltpu.SemaphoreType.DMA,
        ],
    )(src, dec, x, w0r, b2)

    return out


# speedup vs baseline: 1.6483x; 1.2367x over previous
import jax
import jax.numpy as jnp
import numpy as np
from jax import lax
from jax.experimental import pallas as pl
from jax.experimental.pallas import tpu as pltpu
from jax.experimental.pallas import tpu_sc as plsc

_B = 64
_EMBED = 768
_NVIS = 256
_NMASK = 768
_NROWS = 1025
_NW = 32
_RB = 32
_TPB = 32
_SB = 16
_KMAX = (_SB * _TPB) // _NW
_TR = 128
_TGRID = (_NROWS + _TR - 1) // _TR
_NPAD = _TGRID * _TR


def _build_maps(sidx_ref, midx_ref, src_ref):
    def init(i, _):
        src_ref[i] = -1
        return 0

    lax.fori_loop(0, _NPAD, init, 0)
    src_ref[0] = 0

    def samp(j, _):
        src_ref[sidx_ref[j]] = j + 1
        return 0

    lax.fori_loop(0, _NVIS, samp, 0)

    def msk(j, _):
        src_ref[midx_ref[j]] = -2
        return 0

    lax.fori_loop(0, _NMASK, msk, 0)


def _sc_body(x_hbm, src_hbm, w0_hbm, b_hbm, out_hbm,
             srcb, gxb, slotb, xstage, gxb2, slotb2, xstage2,
             stg0, stg1, wbuf, bbuf, semg, semg2, semo):
    cid = lax.axis_index("c")
    sid = lax.axis_index("s")
    wid = sid * 2 + cid
    pltpu.sync_copy(src_hbm, srcb)
    pltpu.sync_copy(w0_hbm, wbuf)
    pltpu.sync_copy(b_hbm.at[pl.ds(0, 16)], bbuf)
    acc = jnp.zeros((16,), jnp.float32)
    for k in range(_EMBED // 16):
        acc = acc + wbuf[pl.ds(k * 16, 16)]
    wsum = acc[0]
    for k in range(1, 16):
        wsum = wsum + acc[k]
    b0 = bbuf[...][0]
    s_val = wsum * np.float32(127.0 / 255.0) + b0
    sv = jnp.full((16,), s_val, jnp.float32)
    zv = jnp.zeros((16,), jnp.float32)

    def start_block(k, stg, gxbk, slotbk, xstagek, sem):
        blkid = wid + _NW * k
        bb = blkid // _TPB
        r0 = lax.rem(blkid, _TPB) * _RB

        def row(i, cur):
            src_i = srcb[pl.ds(r0 + i, 16)][0]

            @pl.when(src_i >= 0)
            def _():
                gxbk[pl.ds(cur, 16)] = jnp.full(
                    (16,), bb * (1 + _NVIS), jnp.int32
                ) + src_i
                slotbk[pl.ds(cur, 16)] = jnp.full((16,), i, jnp.int32)

            @pl.when(src_i == -1)
            def _():
                for c in range(_EMBED // 16):
                    stg[i, pl.ds(c * 16, 16)] = zv

            @pl.when(src_i == -2)
            def _():
                for c in range(_EMBED // 16):
                    stg[i, pl.ds(c * 16, 16)] = sv

            return cur + (src_i >= 0).astype(jnp.int32)

        cur = lax.fori_loop(0, _RB, row, 0)
        nch = (cur + 7) // 8

        def fire(c, _):
            pltpu.async_copy(
                x_hbm.at[gxbk.at[pl.ds(c * 8, 8)]],
                xstagek.at[pl.ds(c * 8, 8)],
                sem,
            )
            return 0

        lax.fori_loop(0, nch, fire, 0)
        return cur

    def finish_block(k, stg, gxbk, slotbk, xstagek, cur, sem):
        blkid = wid + _NW * k
        bb = blkid // _TPB
        r0 = lax.rem(blkid, _TPB) * _RB
        nch = (cur + 7) // 8

        def draing(c, _):
            pltpu.make_async_copy(
                x_hbm.at[gxbk.at[pl.ds(0, 8)]], xstagek.at[pl.ds(0, 8)], sem
            ).wait()
            return 0

        lax.fori_loop(0, nch, draing, 0)

        def redis(m, _):
            slot = slotbk[pl.ds(m, 16)][0]
            for c in range(_EMBED // 16):
                stg[slot, pl.ds(c * 16, 16)] = xstagek[m, pl.ds(c * 16, 16)]
            return 0

        lax.fori_loop(0, cur, redis, 0)

        pltpu.async_copy(stg, out_hbm.at[bb, pl.ds(r0, _RB)], semo)

    def outwait():
        pltpu.make_async_copy(stg0, out_hbm.at[0, pl.ds(0, _RB)], semo).wait()

    def pair_body(kk, cur_prev):
        k0 = 2 * kk
        k1 = k0 + 1

        @pl.when(kk >= 1)
        def _():
            outwait()

        cur0 = start_block(k0, stg0, gxb, slotb, xstage, semg)

        @pl.when(kk >= 1)
        def _():
            finish_block(k0 - 1, stg1, gxb2, slotb2, xstage2, cur_prev, semg2)
            outwait()

        cur1 = start_block(k1, stg1, gxb2, slotb2, xstage2, semg2)
        finish_block(k0, stg0, gxb, slotb, xstage, cur0, semg)
        return cur1

    cur_last = lax.fori_loop(0, _KMAX // 2, pair_body, 0)
    finish_block(_KMAX - 1, stg1, gxb2, slotb2, xstage2, cur_last, semg2)
    outwait()
    outwait()


def _tc_assemble(src_ref, dec_in, x_hbm, w0_ref, b_ref, o_ref, sem, cnt_ref):
    h = pl.program_id(0)
    t = pl.program_id(1)
    s_val = jnp.sum(w0_ref[...]) * np.float32(127.0 / 255.0) + b_ref[0, 0]
    cnt_ref[0] = 0

    def row(i, c):
        src = src_ref[t * _TR + i]

        @pl.when(src >= 0)
        def _():
            pltpu.make_async_copy(
                x_hbm.at[pl.ds(16 * h + _SB, 16), pl.ds(src, 1)],
                o_ref.at[:, pl.ds(i, 1)],
                sem,
            ).start()
            cnt_ref[0] = cnt_ref[0] + 1

        @pl.when(src == -1)
        def _():
            o_ref[:, pl.ds(i, 1)] = jnp.zeros((16, 1, _EMBED), jnp.float32)

        @pl.when(src == -2)
        def _():
            o_ref[:, pl.ds(i, 1)] = jnp.full((16, 1, _EMBED), s_val)

        return c

    lax.fori_loop(0, _TR, row, 0)

    def drain(k, c):
        pltpu.make_async_copy(
            x_hbm.at[pl.ds(_SB, 16), pl.ds(0, 1)],
            o_ref.at[:, pl.ds(0, 1)],
            sem,
        ).wait()
        return c

    lax.fori_loop(0, cnt_ref[0], drain, 0)


def _tail(src_ref, dec_hbm, x_hbm, w0_ref, b_ref, o_hbm, stg, sem):
    src_t = src_ref[_NROWS - 1]
    s_val = jnp.sum(w0_ref[...]) * np.float32(127.0 / 255.0) + b_ref[0, 0]

    @pl.when(src_t >= 0)
    def _():
        cp = pltpu.make_async_copy(
            x_hbm.at[:, pl.ds(jnp.maximum(src_t, 0), 1)], stg, sem
        )
        cp.start()
        cp.wait()

    @pl.when(src_t == -1)
    def _():
        stg[...] = jnp.zeros((_B, 1, _EMBED), jnp.float32)

    @pl.when(src_t == -2)
    def _():
        stg[...] = jnp.full((_B, 1, _EMBED), s_val)

    cp = pltpu.make_async_copy(
        stg, o_hbm.at[:, pl.ds(_NROWS - 1, 1)], sem
    )
    cp.start()
    cp.wait()


def kernel(x, sample_index, mask_index, W, b):
    src = pl.pallas_call(
        _build_maps,
        in_specs=[
            pl.BlockSpec(memory_space=pltpu.SMEM),
            pl.BlockSpec(memory_space=pltpu.SMEM),
        ],
        out_specs=pl.BlockSpec(memory_space=pltpu.SMEM),
        out_shape=jax.ShapeDtypeStruct((_NPAD,), jnp.int32),
    )(sample_index, mask_index)

    x2d = jnp.reshape(x[:_SB], (_SB * (1 + _NVIS), _EMBED))
    w0 = jnp.reshape(W[0], (_EMBED,))

    mesh = plsc.VectorSubcoreMesh(core_axis_name="c", subcore_axis_name="s")
    dec = pl.kernel(
        _sc_body,
        out_type=jax.ShapeDtypeStruct((_B, _NROWS, _EMBED), jnp.float32),
        mesh=mesh,
        scratch_types=[
            pltpu.VMEM((_NPAD,), jnp.int32),
            pltpu.VMEM((64,), jnp.int32),
            pltpu.VMEM((64,), jnp.int32),
            pltpu.VMEM((_RB, _EMBED), jnp.float32),
            pltpu.VMEM((64,), jnp.int32),
            pltpu.VMEM((64,), jnp.int32),
            pltpu.VMEM((_RB, _EMBED), jnp.float32),
            pltpu.VMEM((_RB, _EMBED), jnp.float32),
            pltpu.VMEM((_RB, _EMBED), jnp.float32),
            pltpu.VMEM((_EMBED,), jnp.float32),
            pltpu.VMEM((16,), jnp.float32),
            pltpu.SemaphoreType.DMA,
            pltpu.SemaphoreType.DMA,
            pltpu.SemaphoreType.DMA,
        ],
    )(x2d, src, w0, b)

    w0r = jnp.reshape(W[0], (1, _EMBED))
    b2 = jnp.reshape(b, (1, _EMBED))
    dec = pl.pallas_call(
        _tc_assemble,
        grid=((_B - _SB) // 16, _TGRID),
        in_specs=[
            pl.BlockSpec(memory_space=pltpu.SMEM),
            pl.BlockSpec(memory_space=pl.ANY),
            pl.BlockSpec(memory_space=pl.ANY),
            pl.BlockSpec((1, _EMBED), lambda h, t: (0, 0)),
            pl.BlockSpec((1, _EMBED), lambda h, t: (0, 0)),
        ],
        out_specs=pl.BlockSpec((16, _TR, _EMBED), lambda h, t: (h + 1, t, 0)),
        out_shape=jax.ShapeDtypeStruct((_B, _NROWS, _EMBED), jnp.float32),
        input_output_aliases={1: 0},
        scratch_shapes=[
            pltpu.SemaphoreType.DMA,
            pltpu.SMEM((1,), jnp.int32),
        ],
    )(src, dec, x, w0r, b2)

    out = pl.pallas_call(
        _tail,
        in_specs=[
            pl.BlockSpec(memory_space=pltpu.SMEM),
            pl.BlockSpec(memory_space=pl.ANY),
            pl.BlockSpec(memory_space=pl.ANY),
            pl.BlockSpec((1, _EMBED), lambda: (0, 0)),
            pl.BlockSpec((1, _EMBED), lambda: (0, 0)),
        ],
        out_specs=pl.BlockSpec(memory_space=pl.ANY),
        out_shape=jax.ShapeDtypeStruct((_B, _NROWS, _EMBED), jnp.float32),
        input_output_aliases={1: 0},
        scratch_shapes=[
            pltpu.VMEM((_B, 1, _EMBED), jnp.float32),
            pltpu.SemaphoreType.DMA,
        ],
    )(src, dec, x, w0r, b2)

    return out
